# Initial kernel scaffold; baseline (speedup 1.0000x reference)
import jax
import jax.numpy as jnp
from jax.experimental import pallas as pl


def kernel(z, pos, batch, edge_index, edge_feats, params):
    raise NotImplementedError("write your pallas kernel here")


# speedup vs baseline: 2.3838x; 2.3838x over previous
import jax
import jax.numpy as jnp
from jax import lax
from jax.experimental import pallas as pl
from jax.experimental.pallas import tpu as pltpu
from jax.experimental.pallas import tpu_sc as plsc

N = 10000
E = 320000
D = 128
DE = 16
G = 64
H = 8
DH = 16
FFN = 256
NEURONS = 512
NPAD = 10240

NC, NS = 2, 16
NW = NC * NS
EW = E // NW
ECH = 80
ENCH = EW // ECH
NACC = 10240
NROWS_W = NPAD // NW
NCH = 80
NNCH = NROWS_W // NCH

def _sc_mesh():
    return plsc.VectorSubcoreMesh(
        core_axis_name="c", subcore_axis_name="s",
        num_cores=NC, num_subcores=NS)


def _sig(x):
    return 1.0 / (1.0 + jnp.exp(-x))


def _silu(x):
    return x * _sig(x)


def _sc_prelude(emb, zr, pos16, rowr, colr):
    def body(emb_h, zr_h, pos_h, rowr_h, colr_h, h0_h, pr_h, pc_h,
             idxn_v, idxe_v, bufn_v, bufe_v, sem):
        wid = lax.axis_index("s") * NC + lax.axis_index("c")
        pltpu.sync_copy(zr_h.at[wid], idxn_v)
        for j in range(NNCH):
            pltpu.async_copy(emb_h.at[idxn_v.at[j]], bufn_v, sem).wait()
            pltpu.sync_copy(bufn_v, h0_h.at[pl.ds(wid * NROWS_W + j * NCH, NCH)])

        def gather_pos(idx_src, out_h):
            pltpu.sync_copy(idx_src.at[wid], idxe_v)

            def step(j, carry):
                pltpu.async_copy(pos_h.at[idxe_v.at[j]], bufe_v, sem).wait()
                pltpu.sync_copy(bufe_v, out_h.at[pl.ds(wid * EW + j * ECH, ECH)])
                return carry

            lax.fori_loop(0, ENCH, step, 0)

        gather_pos(rowr_h, pr_h)
        gather_pos(colr_h, pc_h)

    out_type = (jax.ShapeDtypeStruct((NPAD, D), jnp.float32),
                jax.ShapeDtypeStruct((E, DE), jnp.float32),
                jax.ShapeDtypeStruct((E, DE), jnp.float32))
    return pl.kernel(
        body, out_type=out_type, mesh=_sc_mesh(),
        compiler_params=pltpu.CompilerParams(use_tc_tiling_on_sc=False),
        scratch_types=[pltpu.VMEM((NNCH, NCH), jnp.int32),
                       pltpu.VMEM((ENCH, ECH), jnp.int32),
                       pltpu.VMEM((NCH, D), jnp.float32),
                       pltpu.VMEM((ECH, DE), jnp.float32),
                       pltpu.SemaphoreType.DMA],
    )(emb, zr, pos16, rowr, colr)


def _sc_pair_gather(hA, hB, rowr, colr):
    def body(hA_h, hB_h, rowr_h, colr_h, gA_h, gB_h,
             idxa_v, idxb_v, bufa_v, bufb_v, sema, semb):
        wid = lax.axis_index("s") * NC + lax.axis_index("c")
        pltpu.sync_copy(rowr_h.at[wid], idxa_v)
        pltpu.sync_copy(colr_h.at[wid], idxb_v)

        def step(j, carry):
            ca = pltpu.async_copy(hA_h.at[idxa_v.at[j]], bufa_v, sema)
            cb = pltpu.async_copy(hB_h.at[idxb_v.at[j]], bufb_v, semb)
            ca.wait()
            cb.wait()
            base = wid * EW + j * ECH
            pltpu.sync_copy(bufa_v, gA_h.at[pl.ds(base, ECH)])
            pltpu.sync_copy(bufb_v, gB_h.at[pl.ds(base, ECH)])
            return carry

        lax.fori_loop(0, ENCH, step, 0)

    out_type = (jax.ShapeDtypeStruct((E, D), jnp.float32),
                jax.ShapeDtypeStruct((E, D), jnp.float32))
    return pl.kernel(
        body, out_type=out_type, mesh=_sc_mesh(),
        scratch_types=[pltpu.VMEM((ENCH, ECH), jnp.int32),
                       pltpu.VMEM((ENCH, ECH), jnp.int32),
                       pltpu.VMEM((ECH, D), jnp.float32),
                       pltpu.VMEM((ECH, D), jnp.float32),
                       pltpu.SemaphoreType.DMA,
                       pltpu.SemaphoreType.DMA],
    )(hA, hB, rowr, colr)


def _sc_scatter_add(m, rowr, zeros_acc):
    RPT = NACC // NS

    def body(m_h, rowr_h, z_h, out_h, idx_v, buf_v, acc_sh, sem):
        cid = lax.axis_index("c")
        sid = lax.axis_index("s")
        wid = sid * NC + cid
        pltpu.sync_copy(z_h.at[pl.ds(sid * RPT, RPT)],
                        acc_sh.at[pl.ds(sid * RPT, RPT)])
        plsc.subcore_barrier()
        pltpu.sync_copy(rowr_h.at[wid], idx_v)

        def step(j, carry):
            pltpu.async_copy(m_h.at[pl.ds(wid * EW + j * ECH, ECH)],
                             buf_v, sem).wait()
            pltpu.sync_copy(buf_v, acc_sh.at[idx_v.at[j]], add=True)
            return carry

        lax.fori_loop(0, ENCH, step, 0)
        plsc.subcore_barrier()
        pltpu.sync_copy(acc_sh.at[pl.ds(sid * RPT, RPT)],
                        out_h.at[cid, pl.ds(sid * RPT, RPT)])

    return pl.kernel(
        body,
        out_type=jax.ShapeDtypeStruct((NC, NACC, D), jnp.float32),
        mesh=_sc_mesh(),
        scratch_types=[pltpu.VMEM((ENCH, ECH), jnp.int32),
                       pltpu.VMEM((ECH, D), jnp.float32),
                       pltpu.VMEM_SHARED((NACC, D), jnp.float32),
                       pltpu.SemaphoreType.DMA],
    )(m, rowr, zeros_acc)


def _mm_multi(x, ws, bs, nblk):
    R = x.shape[0]
    BL = R // nblk
    nw = len(ws)

    def body(x_ref, *refs):
        w_refs = refs[:nw]
        b_refs = refs[nw:2 * nw]
        o_refs = refs[2 * nw:]
        xb = x_ref[...]
        for wr, br, orf in zip(w_refs, b_refs, o_refs):
            orf[...] = jnp.dot(xb, wr[...],
                               preferred_element_type=jnp.float32) + br[...]

    in_specs = ([pl.BlockSpec((BL, x.shape[1]), lambda i: (i, 0))]
                + [pl.BlockSpec(w.shape, lambda i: (0, 0)) for w in ws]
                + [pl.BlockSpec(b.shape, lambda i: (0, 0)) for b in bs])
    out_specs = [pl.BlockSpec((BL, w.shape[1]), lambda i: (i, 0)) for w in ws]
    out_shape = [jax.ShapeDtypeStruct((R, w.shape[1]), jnp.float32) for w in ws]
    return pl.pallas_call(
        body, grid=(nblk,), in_specs=in_specs, out_specs=out_specs,
        out_shape=out_shape,
    )(x, *ws, *bs)


def _edge_mlp(gA, gB, pr, pc, ef, w_r, w_ef, b1, w2, b2, aw_row, ab):
    nblk = 160
    BL = E // nblk

    def body(ga, gb, prr, pcr, efr, wrr, wefr, b1r, w2r, b2r, awr, abr, out):
        dd = prr[...] - pcr[...]
        radial = jnp.sum(dd * dd, axis=1, keepdims=True)
        pre = (ga[...] + gb[...] + radial * wrr[...]
               + jnp.dot(efr[...], wefr[...],
                         preferred_element_type=jnp.float32) + b1r[...])
        m1 = _silu(pre)
        m2 = _silu(jnp.dot(m1, w2r[...],
                           preferred_element_type=jnp.float32) + b2r[...])
        att = _sig(jnp.sum(m2 * awr[...], axis=1, keepdims=True) + abr[...])
        out[...] = m2 * att

    eb = lambda K: pl.BlockSpec((BL, K), lambda i: (i, 0))
    full = lambda a: pl.BlockSpec(a.shape, lambda i: (0, 0))
    return pl.pallas_call(
        body, grid=(nblk,),
        in_specs=[eb(D), eb(D), eb(DE), eb(DE), eb(DE),
                  full(w_r), full(w_ef), full(b1), full(w2), full(b2),
                  full(aw_row), full(ab)],
        out_specs=eb(D),
        out_shape=jax.ShapeDtypeStruct((E, D), jnp.float32),
    )(gA, gB, pr, pc, ef, w_r, w_ef, b1, w2, b2, aw_row, ab)


def _node_update(h, a0, a1, w1h, w1a, nb1, nw2, nb2):
    nblk = 5
    BL = N // nblk

    def body(hr, a0r, a1r, w1hr, w1ar, b1r, w2r, b2r, out):
        hb = hr[...]
        agg = a0r[...] + a1r[...]
        t = _silu(jnp.dot(hb, w1hr[...], preferred_element_type=jnp.float32)
                  + jnp.dot(agg, w1ar[...], preferred_element_type=jnp.float32)
                  + b1r[...])
        out[...] = hb + jnp.dot(t, w2r[...],
                                preferred_element_type=jnp.float32) + b2r[...]

    nb = pl.BlockSpec((BL, D), lambda i: (i, 0))
    full = lambda a: pl.BlockSpec(a.shape, lambda i: (0, 0))
    return pl.pallas_call(
        body, grid=(nblk,),
        in_specs=[nb, nb, nb, full(w1h), full(w1a), full(nb1),
                  full(nw2), full(nb2)],
        out_specs=nb,
        out_shape=jax.ShapeDtypeStruct((N, D), jnp.float32),
    )(h, a0, a1, w1h, w1a, nb1, nw2, nb2)


def _attention(q3, k3, v3):
    BQ = 128
    nq = NPAD // BQ

    def body(q_ref, k_ref, v_ref, o_ref):
        qb = q_ref[0]
        kb = k_ref[0]
        s = lax.dot_general(qb, kb, (((1,), (1,)), ((), ())),
                            preferred_element_type=jnp.float32) * 0.25
        kidx = lax.broadcasted_iota(jnp.int32, (1, NPAD), 1)
        s = jnp.where(kidx < N, s, -1e30)
        mx = jnp.max(s, axis=1, keepdims=True)
        p = jnp.exp(s - mx)
        p = p / jnp.sum(p, axis=1, keepdims=True)
        o_ref[0] = jnp.dot(p, v_ref[0], preferred_element_type=jnp.float32)

    return pl.pallas_call(
        body, grid=(H, nq),
        in_specs=[pl.BlockSpec((1, BQ, DH), lambda h, i: (h, i, 0)),
                  pl.BlockSpec((1, NPAD, DH), lambda h, i: (h, 0, 0)),
                  pl.BlockSpec((1, NPAD, DH), lambda h, i: (h, 0, 0))],
        out_specs=pl.BlockSpec((1, BQ, DH), lambda h, i: (h, i, 0)),
        out_shape=jax.ShapeDtypeStruct((H, NPAD, DH), jnp.float32),
    )(q3, k3, v3)


def _post_attn(h, at, wo, wob, g1, b1, fw1, fb1, fw2, fb2, g2, b2):
    nblk = 5
    BL = N // nblk

    def ln(x, g, b):
        mu = jnp.mean(x, axis=1, keepdims=True)
        xc = x - mu
        var = jnp.mean(xc * xc, axis=1, keepdims=True)
        return xc * lax.rsqrt(var + 1e-5) * g + b

    def body(hr, ar, wor, wobr, g1r, b1r, fw1r, fb1r, fw2r, fb2r,
             g2r, b2r, out):
        a = jnp.dot(ar[...], wor[...],
                    preferred_element_type=jnp.float32) + wobr[...]
        x = ln(hr[...] + a, g1r[...], b1r[...])
        f = jnp.dot(_silu(jnp.dot(x, fw1r[...],
                                  preferred_element_type=jnp.float32)
                          + fb1r[...]),
                    fw2r[...], preferred_element_type=jnp.float32) + fb2r[...]
        out[...] = ln(x + f, g2r[...], b2r[...])

    nb = pl.BlockSpec((BL, D), lambda i: (i, 0))
    full = lambda a: pl.BlockSpec(a.shape, lambda i: (0, 0))
    return pl.pallas_call(
        body, grid=(nblk,),
        in_specs=[nb, nb, full(wo), full(wob), full(g1), full(b1),
                  full(fw1), full(fb1), full(fw2), full(fb2),
                  full(g2), full(b2)],
        out_specs=nb,
        out_shape=jax.ShapeDtypeStruct((N, D), jnp.float32),
    )(h, at, wo, wob, g1, b1, fw1, fb1, fw2, fb2, g2, b2)


def _pool_final(h, batch2, hw1, hb1, hw2, hb2):
    def body(hr, br, w1r, b1r, w2r, b2r, out):
        gid = lax.broadcasted_iota(jnp.int32, (G, N), 0)
        onehot = jnp.where(gid == br[...], 1.0, 0.0)
        pooled = jnp.dot(onehot, hr[...], preferred_element_type=jnp.float32)
        t = _silu(jnp.dot(pooled, w1r[...],
                          preferred_element_type=jnp.float32) + b1r[...])
        out[...] = jnp.dot(t, w2r[...],
                           preferred_element_type=jnp.float32) + b2r[...]

    full = lambda a: pl.BlockSpec(a.shape, lambda: (0,) * a.ndim)
    return pl.pallas_call(
        body, grid=(),
        in_specs=[full(h), full(batch2), full(hw1), full(hb1),
                  full(hw2), full(hb2)],
        out_specs=pl.BlockSpec((G, 1), lambda: (0, 0)),
        out_shape=jax.ShapeDtypeStruct((G, 1), jnp.float32),
    )(h, batch2, hw1, hb1, hw2, hb2)


def kernel(z, pos, batch, edge_index, edge_feats, params):
    p = params
    z = z.astype(jnp.int32)
    ei = edge_index.astype(jnp.int32)
    rowr = ei[0].reshape(NW, ENCH, ECH)
    colr = ei[1].reshape(NW, ENCH, ECH)
    zpad = jnp.pad(z, (0, NPAD - N)).reshape(NW, NNCH, NCH)
    pos16 = jnp.pad(pos.astype(jnp.float32), ((0, 0), (0, DE - 3)))

    h0p, prg, pcg = _sc_prelude(p['emb'], zpad, pos16, rowr, colr)
    h = h0p[:N]
    zeros_acc = jnp.zeros((NACC, D), jnp.float32)
    zb = jnp.zeros((1, D), jnp.float32)

    for l in range(2):
        ew1 = p['g%d_ew1' % l]
        wA, wB = ew1[:D], ew1[D:2 * D]
        w_r, w_ef = ew1[2 * D:2 * D + 1], ew1[2 * D + 1:]
        hA, hB = _mm_multi(h, [wA, wB], [zb, zb], 5)
        gA, gB = _sc_pair_gather(hA, hB, rowr, colr)
        m = _edge_mlp(gA, gB, prg, pcg, edge_feats,
                      w_r, w_ef, p['g%d_eb1' % l].reshape(1, D),
                      p['g%d_ew2' % l], p['g%d_eb2' % l].reshape(1, D),
                      p['g%d_aw' % l].reshape(1, D),
                      p['g%d_ab' % l].reshape(1, 1))
        agg2 = _sc_scatter_add(m, rowr, zeros_acc)
        nw1 = p['g%d_nw1' % l]
        h = _node_update(h, agg2[0, :N], agg2[1, :N], nw1[:D], nw1[D:],
                         p['g%d_nb1' % l].reshape(1, D), p['g%d_nw2' % l],
                         p['g%d_nb2' % l].reshape(1, D))

    hp = jnp.pad(h, ((0, NPAD - N), (0, 0)))
    q, k, v = _mm_multi(hp, [p['wq'], p['wk'], p['wv']],
                        [p['wq_b'].reshape(1, D), p['wk_b'].reshape(1, D),
                         p['wv_b'].reshape(1, D)], 5)
    q3 = q.reshape(NPAD, H, DH).transpose(1, 0, 2)
    k3 = k.reshape(NPAD, H, DH).transpose(1, 0, 2)
    v3 = v.reshape(NPAD, H, DH).transpose(1, 0, 2)
    o3 = _attention(q3, k3, v3)
    at = o3.transpose(1, 0, 2).reshape(NPAD, D)[:N]

    h2 = _post_attn(h, at, p['wo'], p['wo_b'].reshape(1, D),
                    p['ln1_g'].reshape(1, D), p['ln1_b'].reshape(1, D),
                    p['fw1'], p['fb1'].reshape(1, FFN),
                    p['fw2'], p['fb2'].reshape(1, D),
                    p['ln2_g'].reshape(1, D), p['ln2_b'].reshape(1, D))

    return _pool_final(h2, batch.astype(jnp.int32).reshape(1, N),
                       p['hw1'], p['hb1'].reshape(1, NEURONS),
                       p['hw2'], p['hb2'].reshape(1, 1))


# speedup vs baseline: 2.9621x; 1.2426x over previous
import jax
import jax.numpy as jnp
from jax import lax
from jax.experimental import pallas as pl
from jax.experimental.pallas import tpu as pltpu
from jax.experimental.pallas import tpu_sc as plsc

N = 10000
E = 320000
D = 128
DE = 16
G = 64
H = 8
DH = 16
FFN = 256
NEURONS = 512
NPAD = 10240

NC, NS = 2, 16
NW = NC * NS
EW = E // NW
ECH = 80
ENCH = EW // ECH
NACC = 10240
NROWS_W = NPAD // NW
NCH = 80
NNCH = NROWS_W // NCH

def _sc_mesh():
    return plsc.VectorSubcoreMesh(
        core_axis_name="c", subcore_axis_name="s",
        num_cores=NC, num_subcores=NS)


def _sig(x):
    return 1.0 / (1.0 + jnp.exp(-x))


def _silu(x):
    return x * _sig(x)


def _sc_prelude(emb, zr, pos16, rowr, colr):
    def body(emb_h, zr_h, pos_h, rowr_h, colr_h, h0_h, pr_h, pc_h,
             idxn_v, idxe_v, bufn_v, bufe_v, sem):
        wid = lax.axis_index("s") * NC + lax.axis_index("c")
        pltpu.sync_copy(zr_h.at[wid], idxn_v)
        for j in range(NNCH):
            pltpu.async_copy(emb_h.at[idxn_v.at[j]], bufn_v, sem).wait()
            pltpu.sync_copy(bufn_v, h0_h.at[pl.ds(wid * NROWS_W + j * NCH, NCH)])

        def gather_pos(idx_src, out_h):
            pltpu.sync_copy(idx_src.at[wid], idxe_v)

            def step(j, carry):
                pltpu.async_copy(pos_h.at[idxe_v.at[j]], bufe_v, sem).wait()
                pltpu.sync_copy(bufe_v, out_h.at[pl.ds(wid * EW + j * ECH, ECH)])
                return carry

            lax.fori_loop(0, ENCH, step, 0)

        gather_pos(rowr_h, pr_h)
        gather_pos(colr_h, pc_h)

    out_type = (jax.ShapeDtypeStruct((NPAD, D), jnp.float32),
                jax.ShapeDtypeStruct((E, DE), jnp.float32),
                jax.ShapeDtypeStruct((E, DE), jnp.float32))
    return pl.kernel(
        body, out_type=out_type, mesh=_sc_mesh(),
        compiler_params=pltpu.CompilerParams(use_tc_tiling_on_sc=False),
        scratch_types=[pltpu.VMEM((NNCH, NCH), jnp.int32),
                       pltpu.VMEM((ENCH, ECH), jnp.int32),
                       pltpu.VMEM((NCH, D), jnp.float32),
                       pltpu.VMEM((ECH, DE), jnp.float32),
                       pltpu.SemaphoreType.DMA],
    )(emb, zr, pos16, rowr, colr)


def _sc_pair_gather(hA, hB, rowr, colr):
    def body(hA_h, hB_h, rowr_h, colr_h, gA_h, gB_h,
             idxa_v, idxb_v, bufa_v, bufb_v, sema, semb):
        wid = lax.axis_index("s") * NC + lax.axis_index("c")
        pltpu.sync_copy(rowr_h.at[wid], idxa_v)
        pltpu.sync_copy(colr_h.at[wid], idxb_v)

        def step(j, carry):
            ca = pltpu.async_copy(hA_h.at[idxa_v.at[j]], bufa_v, sema)
            cb = pltpu.async_copy(hB_h.at[idxb_v.at[j]], bufb_v, semb)
            ca.wait()
            cb.wait()
            base = wid * EW + j * ECH
            pltpu.sync_copy(bufa_v, gA_h.at[pl.ds(base, ECH)])
            pltpu.sync_copy(bufb_v, gB_h.at[pl.ds(base, ECH)])
            return carry

        lax.fori_loop(0, ENCH, step, 0)

    out_type = (jax.ShapeDtypeStruct((E, D), jnp.float32),
                jax.ShapeDtypeStruct((E, D), jnp.float32))
    return pl.kernel(
        body, out_type=out_type, mesh=_sc_mesh(),
        scratch_types=[pltpu.VMEM((ENCH, ECH), jnp.int32),
                       pltpu.VMEM((ENCH, ECH), jnp.int32),
                       pltpu.VMEM((ECH, D), jnp.float32),
                       pltpu.VMEM((ECH, D), jnp.float32),
                       pltpu.SemaphoreType.DMA,
                       pltpu.SemaphoreType.DMA],
    )(hA, hB, rowr, colr)


def _sc_scatter_add(m, rowr, zeros_acc):
    RPT = NACC // NS

    def body(m_h, rowr_h, z_h, out_h, idx_v, buf_v, acc_sh, sem):
        cid = lax.axis_index("c")
        sid = lax.axis_index("s")
        wid = sid * NC + cid
        pltpu.sync_copy(z_h.at[pl.ds(sid * RPT, RPT)],
                        acc_sh.at[pl.ds(sid * RPT, RPT)])
        plsc.subcore_barrier()
        pltpu.sync_copy(rowr_h.at[wid], idx_v)

        def step(j, carry):
            pltpu.async_copy(m_h.at[pl.ds(wid * EW + j * ECH, ECH)],
                             buf_v, sem).wait()
            pltpu.sync_copy(buf_v, acc_sh.at[idx_v.at[j]], add=True)
            return carry

        lax.fori_loop(0, ENCH, step, 0)
        plsc.subcore_barrier()
        pltpu.sync_copy(acc_sh.at[pl.ds(sid * RPT, RPT)],
                        out_h.at[cid, pl.ds(sid * RPT, RPT)])

    return pl.kernel(
        body,
        out_type=jax.ShapeDtypeStruct((NC, NACC, D), jnp.float32),
        mesh=_sc_mesh(),
        scratch_types=[pltpu.VMEM((ENCH, ECH), jnp.int32),
                       pltpu.VMEM((ECH, D), jnp.float32),
                       pltpu.VMEM_SHARED((NACC, D), jnp.float32),
                       pltpu.SemaphoreType.DMA],
    )(m, rowr, zeros_acc)


def _mm_multi(x, ws, bs, nblk):
    R = x.shape[0]
    BL = R // nblk
    nw = len(ws)

    def body(x_ref, *refs):
        w_refs = refs[:nw]
        b_refs = refs[nw:2 * nw]
        o_refs = refs[2 * nw:]
        xb = x_ref[...]
        for wr, br, orf in zip(w_refs, b_refs, o_refs):
            orf[...] = jnp.dot(xb, wr[...],
                               preferred_element_type=jnp.float32) + br[...]

    in_specs = ([pl.BlockSpec((BL, x.shape[1]), lambda i: (i, 0))]
                + [pl.BlockSpec(w.shape, lambda i: (0, 0)) for w in ws]
                + [pl.BlockSpec(b.shape, lambda i: (0, 0)) for b in bs])
    out_specs = [pl.BlockSpec((BL, w.shape[1]), lambda i: (i, 0)) for w in ws]
    out_shape = [jax.ShapeDtypeStruct((R, w.shape[1]), jnp.float32) for w in ws]
    return pl.pallas_call(
        body, grid=(nblk,), in_specs=in_specs, out_specs=out_specs,
        out_shape=out_shape,
    )(x, *ws, *bs)


def _edge_mlp(gA, gB, pr, pc, ef, w_r, w_ef, b1, w2, b2, aw_row, ab):
    nblk = 160
    BL = E // nblk

    def body(ga, gb, prr, pcr, efr, wrr, wefr, b1r, w2r, b2r, awr, abr, out):
        dd = prr[...] - pcr[...]
        radial = jnp.sum(dd * dd, axis=1, keepdims=True)
        pre = (ga[...] + gb[...] + radial * wrr[...]
               + jnp.dot(efr[...], wefr[...],
                         preferred_element_type=jnp.float32) + b1r[...])
        m1 = _silu(pre)
        m2 = _silu(jnp.dot(m1, w2r[...],
                           preferred_element_type=jnp.float32) + b2r[...])
        att = _sig(jnp.sum(m2 * awr[...], axis=1, keepdims=True) + abr[...])
        out[...] = m2 * att

    eb = lambda K: pl.BlockSpec((BL, K), lambda i: (i, 0))
    full = lambda a: pl.BlockSpec(a.shape, lambda i: (0, 0))
    return pl.pallas_call(
        body, grid=(nblk,),
        in_specs=[eb(D), eb(D), eb(DE), eb(DE), eb(DE),
                  full(w_r), full(w_ef), full(b1), full(w2), full(b2),
                  full(aw_row), full(ab)],
        out_specs=eb(D),
        out_shape=jax.ShapeDtypeStruct((E, D), jnp.float32),
    )(gA, gB, pr, pc, ef, w_r, w_ef, b1, w2, b2, aw_row, ab)


def _node_proj(h, a0, a1, w1h, w1a, nb1, nw2, nb2, ews, ebs):
    nblk = 5
    BL = N // nblk
    ne = len(ews)

    def body(hr, a0r, a1r, w1hr, w1ar, b1r, w2r, b2r, *refs):
        ew_refs = refs[:ne]
        eb_refs = refs[ne:2 * ne]
        hout = refs[2 * ne]
        eouts = refs[2 * ne + 1:]
        hb = hr[...]
        agg = a0r[...] + a1r[...]
        t = _silu(jnp.dot(hb, w1hr[...], preferred_element_type=jnp.float32)
                  + jnp.dot(agg, w1ar[...], preferred_element_type=jnp.float32)
                  + b1r[...])
        hn = hb + jnp.dot(t, w2r[...],
                          preferred_element_type=jnp.float32) + b2r[...]
        hout[...] = hn
        for wr, br, orf in zip(ew_refs, eb_refs, eouts):
            orf[...] = jnp.dot(hn, wr[...],
                               preferred_element_type=jnp.float32) + br[...]

    nb = pl.BlockSpec((BL, D), lambda i: (i, 0))
    full = lambda a: pl.BlockSpec(a.shape, lambda i: (0, 0))
    return pl.pallas_call(
        body, grid=(nblk,),
        in_specs=([nb, nb, nb, full(w1h), full(w1a), full(nb1),
                   full(nw2), full(nb2)]
                  + [full(w) for w in ews] + [full(b) for b in ebs]),
        out_specs=[nb] + [pl.BlockSpec((BL, w.shape[1]), lambda i: (i, 0))
                          for w in ews],
        out_shape=([jax.ShapeDtypeStruct((N, D), jnp.float32)]
                   + [jax.ShapeDtypeStruct((N, w.shape[1]), jnp.float32)
                      for w in ews]),
    )(h, a0, a1, w1h, w1a, nb1, nw2, nb2, *ews, *ebs)


def _attention(q3, k3, v3):
    BQ = 256
    nq = NPAD // BQ

    def body(q_ref, k_ref, v_ref, o_ref):
        s = lax.dot_general(q_ref[0], k_ref[0], (((1,), (1,)), ((), ())),
                            preferred_element_type=jnp.float32)
        p = jnp.exp(s)
        denom = jnp.sum(p, axis=1, keepdims=True) - jnp.float32(NPAD - N)
        o_ref[0] = jnp.dot(p, v_ref[0],
                           preferred_element_type=jnp.float32) / denom

    return pl.pallas_call(
        body, grid=(H, nq),
        in_specs=[pl.BlockSpec((1, BQ, DH), lambda h, i: (h, i, 0)),
                  pl.BlockSpec((1, NPAD, DH), lambda h, i: (h, 0, 0)),
                  pl.BlockSpec((1, NPAD, DH), lambda h, i: (h, 0, 0))],
        out_specs=pl.BlockSpec((1, BQ, DH), lambda h, i: (h, i, 0)),
        out_shape=jax.ShapeDtypeStruct((H, NPAD, DH), jnp.float32),
    )(q3, k3, v3)


def _post_attn_pool(h, at, batch3, wo, wob, g1, b1, fw1, fb1, fw2, fb2,
                    g2, b2, hw1, hb1, hw2, hb2):
    nblk = 5
    BL = N // nblk

    def ln(x, g, b):
        mu = jnp.mean(x, axis=1, keepdims=True)
        xc = x - mu
        var = jnp.mean(xc * xc, axis=1, keepdims=True)
        return xc * lax.rsqrt(var + 1e-5) * g + b

    def body(hr, ar, br3, wor, wobr, g1r, b1r, fw1r, fb1r, fw2r, fb2r,
             g2r, b2r, w1r, bb1r, w2r, bb2r, out, acc):
        i = pl.program_id(0)
        a = jnp.dot(ar[...], wor[...],
                    preferred_element_type=jnp.float32) + wobr[...]
        x = ln(hr[...] + a, g1r[...], b1r[...])
        f = jnp.dot(_silu(jnp.dot(x, fw1r[...],
                                  preferred_element_type=jnp.float32)
                          + fb1r[...]),
                    fw2r[...], preferred_element_type=jnp.float32) + fb2r[...]
        y = ln(x + f, g2r[...], b2r[...])
        gid = lax.broadcasted_iota(jnp.int32, (G, BL), 0)
        onehot = jnp.where(gid == br3[0], 1.0, 0.0)
        part = jnp.dot(onehot, y, preferred_element_type=jnp.float32)

        @pl.when(i == 0)
        def _():
            acc[...] = jnp.zeros_like(acc)

        acc[...] += part

        @pl.when(i == nblk - 1)
        def _():
            t = _silu(jnp.dot(acc[...], w1r[...],
                              preferred_element_type=jnp.float32) + bb1r[...])
            out[...] = jnp.dot(t, w2r[...],
                               preferred_element_type=jnp.float32) + bb2r[...]

    nb = pl.BlockSpec((BL, D), lambda i: (i, 0))
    full = lambda a: pl.BlockSpec(a.shape, lambda i: (0,) * a.ndim)
    return pl.pallas_call(
        body, grid=(nblk,),
        in_specs=[nb, nb, pl.BlockSpec((1, 1, BL), lambda i: (i, 0, 0)),
                  full(wo), full(wob), full(g1), full(b1),
                  full(fw1), full(fb1), full(fw2), full(fb2),
                  full(g2), full(b2), full(hw1), full(hb1),
                  full(hw2), full(hb2)],
        out_specs=pl.BlockSpec((G, 1), lambda i: (0, 0)),
        out_shape=jax.ShapeDtypeStruct((G, 1), jnp.float32),
        scratch_shapes=[pltpu.VMEM((G, D), jnp.float32)],
    )(h, at, batch3, wo, wob, g1, b1, fw1, fb1, fw2, fb2, g2, b2,
      hw1, hb1, hw2, hb2)


def kernel(z, pos, batch, edge_index, edge_feats, params):
    p = params
    z = z.astype(jnp.int32)
    ei = edge_index.astype(jnp.int32)
    rowr = ei[0].reshape(NW, ENCH, ECH)
    colr = ei[1].reshape(NW, ENCH, ECH)
    zpad = jnp.pad(z, (0, NPAD - N)).reshape(NW, NNCH, NCH)
    pos16 = jnp.pad(pos.astype(jnp.float32), ((0, 0), (0, DE - 3)))

    h0p, prg, pcg = _sc_prelude(p['emb'], zpad, pos16, rowr, colr)
    h = h0p[:N]
    zeros_acc = jnp.zeros((NACC, D), jnp.float32)
    zb = jnp.zeros((1, D), jnp.float32)

    def edge_layer(l, hA, hB):
        gA, gB = _sc_pair_gather(hA, hB, rowr, colr)
        ew1 = p['g%d_ew1' % l]
        m = _edge_mlp(gA, gB, prg, pcg, edge_feats,
                      ew1[2 * D:2 * D + 1], ew1[2 * D + 1:],
                      p['g%d_eb1' % l].reshape(1, D),
                      p['g%d_ew2' % l], p['g%d_eb2' % l].reshape(1, D),
                      p['g%d_aw' % l].reshape(1, D),
                      p['g%d_ab' % l].reshape(1, 1))
        return _sc_scatter_add(m, rowr, zeros_acc)

    ew1_0, ew1_1 = p['g0_ew1'], p['g1_ew1']
    hA0, hB0 = _mm_multi(h, [ew1_0[:D], ew1_0[D:2 * D]], [zb, zb], 5)
    agg2 = edge_layer(0, hA0, hB0)
    nw1 = p['g0_nw1']
    h, hA1, hB1 = _node_proj(h, agg2[0, :N], agg2[1, :N], nw1[:D], nw1[D:],
                             p['g0_nb1'].reshape(1, D), p['g0_nw2'],
                             p['g0_nb2'].reshape(1, D),
                             [ew1_1[:D], ew1_1[D:2 * D]], [zb, zb])
    agg2 = edge_layer(1, hA1, hB1)
    nw1 = p['g1_nw1']
    h, q, k, v = _node_proj(h, agg2[0, :N], agg2[1, :N], nw1[:D], nw1[D:],
                            p['g1_nb1'].reshape(1, D), p['g1_nw2'],
                            p['g1_nb2'].reshape(1, D),
                            [p['wq'] * 0.25, p['wk'], p['wv']],
                            [p['wq_b'].reshape(1, D) * 0.25,
                             p['wk_b'].reshape(1, D),
                             p['wv_b'].reshape(1, D)])

    padn = ((0, NPAD - N), (0, 0))
    q3 = jnp.pad(q, padn).reshape(NPAD, H, DH).transpose(1, 0, 2)
    k3 = jnp.pad(k, padn).reshape(NPAD, H, DH).transpose(1, 0, 2)
    v3 = jnp.pad(v, padn).reshape(NPAD, H, DH).transpose(1, 0, 2)
    o3 = _attention(q3, k3, v3)
    at = o3.transpose(1, 0, 2).reshape(NPAD, D)[:N]

    return _post_attn_pool(
        h, at, batch.astype(jnp.int32).reshape(5, 1, N // 5),
        p['wo'], p['wo_b'].reshape(1, D),
        p['ln1_g'].reshape(1, D), p['ln1_b'].reshape(1, D),
        p['fw1'], p['fb1'].reshape(1, FFN),
        p['fw2'], p['fb2'].reshape(1, D),
        p['ln2_g'].reshape(1, D), p['ln2_b'].reshape(1, D),
        p['hw1'], p['hb1'].reshape(1, NEURONS),
        p['hw2'], p['hb2'].reshape(1, 1))


# speedup vs baseline: 3.2908x; 1.1110x over previous
import jax
import jax.numpy as jnp
from jax import lax
from jax.experimental import pallas as pl
from jax.experimental.pallas import tpu as pltpu
from jax.experimental.pallas import tpu_sc as plsc

N = 10000
E = 320000
D = 128
DE = 16
G = 64
H = 8
DH = 16
FFN = 256
NEURONS = 512
NPAD = 10240

NC, NS = 2, 16
NW = NC * NS
EW = E // NW
ECH = 80
ENCH = EW // ECH
NACC = 10240
NROWS_W = NPAD // NW
NCH = 80
NNCH = NROWS_W // NCH

def _sc_mesh():
    return plsc.VectorSubcoreMesh(
        core_axis_name="c", subcore_axis_name="s",
        num_cores=NC, num_subcores=NS)


def _sig(x):
    return 1.0 / (1.0 + jnp.exp(-x))


def _silu(x):
    return x * _sig(x)


def _sc_prelude(emb, zr, pos16, rowr, colr):
    def body(emb_h, zr_h, pos_h, rowr_h, colr_h, h0_h, pr_h, pc_h,
             idxn_v, idxe_v, bufn_v, bufe_v, sem, sem2):
        wid = lax.axis_index("s") * NC + lax.axis_index("c")
        pltpu.sync_copy(zr_h.at[wid], idxn_v)
        for j in range(NNCH):
            pltpu.async_copy(emb_h.at[idxn_v.at[j]], bufn_v, sem).wait()
            pltpu.sync_copy(bufn_v, h0_h.at[pl.ds(wid * NROWS_W + j * NCH, NCH)])

        def gather_pos(idx_src, out_h):
            pltpu.sync_copy(idx_src.at[wid], idxe_v)
            pltpu.async_copy(pos_h.at[idxe_v.at[0]], bufe_v.at[0], sem2.at[0])

            def step(j, carry):
                slot = lax.rem(j, 2)
                nslot = 1 - slot

                @pl.when(j + 1 < ENCH)
                def _():
                    pltpu.async_copy(pos_h.at[idxe_v.at[j + 1]],
                                     bufe_v.at[nslot], sem2.at[nslot])

                pltpu.make_async_copy(pos_h.at[idxe_v.at[j]],
                                      bufe_v.at[slot], sem2.at[slot]).wait()
                pltpu.sync_copy(bufe_v.at[slot],
                                out_h.at[pl.ds(wid * EW + j * ECH, ECH)])
                return carry

            lax.fori_loop(0, ENCH, step, 0)

        gather_pos(rowr_h, pr_h)
        gather_pos(colr_h, pc_h)

    out_type = (jax.ShapeDtypeStruct((NPAD, D), jnp.float32),
                jax.ShapeDtypeStruct((E, DE), jnp.float32),
                jax.ShapeDtypeStruct((E, DE), jnp.float32))
    return pl.kernel(
        body, out_type=out_type, mesh=_sc_mesh(),
        compiler_params=pltpu.CompilerParams(use_tc_tiling_on_sc=False),
        scratch_types=[pltpu.VMEM((NNCH, NCH), jnp.int32),
                       pltpu.VMEM((ENCH, ECH), jnp.int32),
                       pltpu.VMEM((NCH, D), jnp.float32),
                       pltpu.VMEM((2, ECH, DE), jnp.float32),
                       pltpu.SemaphoreType.DMA,
                       pltpu.SemaphoreType.DMA((2,))],
    )(emb, zr, pos16, rowr, colr)


def _sc_pair_gather(hA, hB, rowr, colr):
    def body(hA_h, hB_h, rowr_h, colr_h, gA_h, gB_h,
             idxa_v, idxb_v, bufa_v, bufb_v, sema, semb):
        wid = lax.axis_index("s") * NC + lax.axis_index("c")
        pltpu.sync_copy(rowr_h.at[wid], idxa_v)
        pltpu.sync_copy(colr_h.at[wid], idxb_v)

        def fire(j, slot):
            pltpu.async_copy(hA_h.at[idxa_v.at[j]], bufa_v.at[slot],
                             sema.at[slot])
            pltpu.async_copy(hB_h.at[idxb_v.at[j]], bufb_v.at[slot],
                             semb.at[slot])

        fire(0, 0)

        def step(j, carry):
            slot = lax.rem(j, 2)
            nslot = 1 - slot

            @pl.when(j + 1 < ENCH)
            def _():
                fire(j + 1, nslot)

            pltpu.make_async_copy(hA_h.at[idxa_v.at[j]], bufa_v.at[slot],
                                  sema.at[slot]).wait()
            pltpu.make_async_copy(hB_h.at[idxb_v.at[j]], bufb_v.at[slot],
                                  semb.at[slot]).wait()
            base = wid * EW + j * ECH
            pltpu.sync_copy(bufa_v.at[slot], gA_h.at[pl.ds(base, ECH)])
            pltpu.sync_copy(bufb_v.at[slot], gB_h.at[pl.ds(base, ECH)])
            return carry

        lax.fori_loop(0, ENCH, step, 0)

    out_type = (jax.ShapeDtypeStruct((E, D), jnp.float32),
                jax.ShapeDtypeStruct((E, D), jnp.float32))
    return pl.kernel(
        body, out_type=out_type, mesh=_sc_mesh(),
        scratch_types=[pltpu.VMEM((ENCH, ECH), jnp.int32),
                       pltpu.VMEM((ENCH, ECH), jnp.int32),
                       pltpu.VMEM((2, ECH, D), jnp.float32),
                       pltpu.VMEM((2, ECH, D), jnp.float32),
                       pltpu.SemaphoreType.DMA((2,)),
                       pltpu.SemaphoreType.DMA((2,))],
    )(hA, hB, rowr, colr)


def _sc_scatter_add(m, rowr, zeros_acc):
    RPT = NACC // NS

    def body(m_h, rowr_h, z_h, out_h, idx_v, buf_v, acc_sh, sem):
        cid = lax.axis_index("c")
        sid = lax.axis_index("s")
        wid = sid * NC + cid
        pltpu.sync_copy(z_h.at[pl.ds(sid * RPT, RPT)],
                        acc_sh.at[pl.ds(sid * RPT, RPT)])
        plsc.subcore_barrier()
        pltpu.sync_copy(rowr_h.at[wid], idx_v)

        def fire(j, slot):
            pltpu.async_copy(m_h.at[pl.ds(wid * EW + j * ECH, ECH)],
                             buf_v.at[slot], sem.at[slot])

        fire(0, 0)

        def step(j, carry):
            slot = lax.rem(j, 2)
            nslot = 1 - slot

            @pl.when(j + 1 < ENCH)
            def _():
                fire(j + 1, nslot)

            pltpu.make_async_copy(m_h.at[pl.ds(wid * EW + j * ECH, ECH)],
                                  buf_v.at[slot], sem.at[slot]).wait()
            pltpu.sync_copy(buf_v.at[slot], acc_sh.at[idx_v.at[j]], add=True)
            return carry

        lax.fori_loop(0, ENCH, step, 0)
        plsc.subcore_barrier()
        pltpu.sync_copy(acc_sh.at[pl.ds(sid * RPT, RPT)],
                        out_h.at[cid, pl.ds(sid * RPT, RPT)])

    return pl.kernel(
        body,
        out_type=jax.ShapeDtypeStruct((NC, NACC, D), jnp.float32),
        mesh=_sc_mesh(),
        scratch_types=[pltpu.VMEM((ENCH, ECH), jnp.int32),
                       pltpu.VMEM((2, ECH, D), jnp.float32),
                       pltpu.VMEM_SHARED((NACC, D), jnp.float32),
                       pltpu.SemaphoreType.DMA((2,))],
    )(m, rowr, zeros_acc)


def _mm_multi(x, ws, bs, nblk):
    R = x.shape[0]
    BL = R // nblk
    nw = len(ws)

    def body(x_ref, *refs):
        w_refs = refs[:nw]
        b_refs = refs[nw:2 * nw]
        o_refs = refs[2 * nw:]
        xb = x_ref[...]
        for wr, br, orf in zip(w_refs, b_refs, o_refs):
            orf[...] = jnp.dot(xb, wr[...],
                               preferred_element_type=jnp.float32) + br[...]

    in_specs = ([pl.BlockSpec((BL, x.shape[1]), lambda i: (i, 0))]
                + [pl.BlockSpec(w.shape, lambda i: (0, 0)) for w in ws]
                + [pl.BlockSpec(b.shape, lambda i: (0, 0)) for b in bs])
    out_specs = [pl.BlockSpec((BL, w.shape[1]), lambda i: (i, 0)) for w in ws]
    out_shape = [jax.ShapeDtypeStruct((R, w.shape[1]), jnp.float32) for w in ws]
    return pl.pallas_call(
        body, grid=(nblk,), in_specs=in_specs, out_specs=out_specs,
        out_shape=out_shape,
    )(x, *ws, *bs)


def _edge_mlp(gA, gB, pr, pc, ef, w_r, w_ef, b1, w2, b2, aw_row, ab):
    nblk = 160
    BL = E // nblk

    def body(ga, gb, prr, pcr, efr, wrr, wefr, b1r, w2r, b2r, awr, abr, out):
        dd = prr[...] - pcr[...]
        radial = jnp.sum(dd * dd, axis=1, keepdims=True)
        pre = (ga[...] + gb[...] + radial * wrr[...]
               + jnp.dot(efr[...], wefr[...],
                         preferred_element_type=jnp.float32) + b1r[...])
        m1 = _silu(pre)
        m2 = _silu(jnp.dot(m1, w2r[...],
                           preferred_element_type=jnp.float32) + b2r[...])
        att = _sig(jnp.sum(m2 * awr[...], axis=1, keepdims=True) + abr[...])
        out[...] = m2 * att

    eb = lambda K: pl.BlockSpec((BL, K), lambda i: (i, 0))
    full = lambda a: pl.BlockSpec(a.shape, lambda i: (0, 0))
    return pl.pallas_call(
        body, grid=(nblk,),
        in_specs=[eb(D), eb(D), eb(DE), eb(DE), eb(DE),
                  full(w_r), full(w_ef), full(b1), full(w2), full(b2),
                  full(aw_row), full(ab)],
        out_specs=eb(D),
        out_shape=jax.ShapeDtypeStruct((E, D), jnp.float32),
    )(gA, gB, pr, pc, ef, w_r, w_ef, b1, w2, b2, aw_row, ab)


def _node_proj(h, a0, a1, w1h, w1a, nb1, nw2, nb2, ews, ebs):
    nblk = 5
    BL = N // nblk
    ne = len(ews)

    def body(hr, a0r, a1r, w1hr, w1ar, b1r, w2r, b2r, *refs):
        ew_refs = refs[:ne]
        eb_refs = refs[ne:2 * ne]
        hout = refs[2 * ne]
        eouts = refs[2 * ne + 1:]
        hb = hr[...]
        agg = a0r[...] + a1r[...]
        t = _silu(jnp.dot(hb, w1hr[...], preferred_element_type=jnp.float32)
                  + jnp.dot(agg, w1ar[...], preferred_element_type=jnp.float32)
                  + b1r[...])
        hn = hb + jnp.dot(t, w2r[...],
                          preferred_element_type=jnp.float32) + b2r[...]
        hout[...] = hn
        for wr, br, orf in zip(ew_refs, eb_refs, eouts):
            orf[...] = jnp.dot(hn, wr[...],
                               preferred_element_type=jnp.float32) + br[...]

    nb = pl.BlockSpec((BL, D), lambda i: (i, 0))
    full = lambda a: pl.BlockSpec(a.shape, lambda i: (0, 0))
    return pl.pallas_call(
        body, grid=(nblk,),
        in_specs=([nb, nb, nb, full(w1h), full(w1a), full(nb1),
                   full(nw2), full(nb2)]
                  + [full(w) for w in ews] + [full(b) for b in ebs]),
        out_specs=[nb] + [pl.BlockSpec((BL, w.shape[1]), lambda i: (i, 0))
                          for w in ews],
        out_shape=([jax.ShapeDtypeStruct((N, D), jnp.float32)]
                   + [jax.ShapeDtypeStruct((N, w.shape[1]), jnp.float32)
                      for w in ews]),
    )(h, a0, a1, w1h, w1a, nb1, nw2, nb2, *ews, *ebs)


def _attention(q3, k3, v3):
    BQ = 256
    nq = NPAD // BQ

    def body(q_ref, k_ref, v_ref, o_ref):
        s = lax.dot_general(q_ref[0], k_ref[0], (((1,), (1,)), ((), ())),
                            preferred_element_type=jnp.float32)
        p = jnp.exp(s)
        denom = jnp.sum(p, axis=1, keepdims=True) - jnp.float32(NPAD - N)
        o_ref[0] = jnp.dot(p, v_ref[0],
                           preferred_element_type=jnp.float32) / denom

    return pl.pallas_call(
        body, grid=(H, nq),
        in_specs=[pl.BlockSpec((1, BQ, DH), lambda h, i: (h, i, 0)),
                  pl.BlockSpec((1, NPAD, DH), lambda h, i: (h, 0, 0)),
                  pl.BlockSpec((1, NPAD, DH), lambda h, i: (h, 0, 0))],
        out_specs=pl.BlockSpec((1, BQ, DH), lambda h, i: (h, i, 0)),
        out_shape=jax.ShapeDtypeStruct((H, NPAD, DH), jnp.float32),
    )(q3, k3, v3)


def _post_attn_pool(h, at, batch3, wo, wob, g1, b1, fw1, fb1, fw2, fb2,
                    g2, b2, hw1, hb1, hw2, hb2):
    nblk = 5
    BL = N // nblk

    def ln(x, g, b):
        mu = jnp.mean(x, axis=1, keepdims=True)
        xc = x - mu
        var = jnp.mean(xc * xc, axis=1, keepdims=True)
        return xc * lax.rsqrt(var + 1e-5) * g + b

    def body(hr, ar, br3, wor, wobr, g1r, b1r, fw1r, fb1r, fw2r, fb2r,
             g2r, b2r, w1r, bb1r, w2r, bb2r, out, acc):
        i = pl.program_id(0)
        a = jnp.dot(ar[...], wor[...],
                    preferred_element_type=jnp.float32) + wobr[...]
        x = ln(hr[...] + a, g1r[...], b1r[...])
        f = jnp.dot(_silu(jnp.dot(x, fw1r[...],
                                  preferred_element_type=jnp.float32)
                          + fb1r[...]),
                    fw2r[...], preferred_element_type=jnp.float32) + fb2r[...]
        y = ln(x + f, g2r[...], b2r[...])
        gid = lax.broadcasted_iota(jnp.int32, (G, BL), 0)
        onehot = jnp.where(gid == br3[0], 1.0, 0.0)
        part = jnp.dot(onehot, y, preferred_element_type=jnp.float32)

        @pl.when(i == 0)
        def _():
            acc[...] = jnp.zeros_like(acc)

        acc[...] += part

        @pl.when(i == nblk - 1)
        def _():
            t = _silu(jnp.dot(acc[...], w1r[...],
                              preferred_element_type=jnp.float32) + bb1r[...])
            out[...] = jnp.dot(t, w2r[...],
                               preferred_element_type=jnp.float32) + bb2r[...]

    nb = pl.BlockSpec((BL, D), lambda i: (i, 0))
    full = lambda a: pl.BlockSpec(a.shape, lambda i: (0,) * a.ndim)
    return pl.pallas_call(
        body, grid=(nblk,),
        in_specs=[nb, nb, pl.BlockSpec((1, 1, BL), lambda i: (i, 0, 0)),
                  full(wo), full(wob), full(g1), full(b1),
                  full(fw1), full(fb1), full(fw2), full(fb2),
                  full(g2), full(b2), full(hw1), full(hb1),
                  full(hw2), full(hb2)],
        out_specs=pl.BlockSpec((G, 1), lambda i: (0, 0)),
        out_shape=jax.ShapeDtypeStruct((G, 1), jnp.float32),
        scratch_shapes=[pltpu.VMEM((G, D), jnp.float32)],
    )(h, at, batch3, wo, wob, g1, b1, fw1, fb1, fw2, fb2, g2, b2,
      hw1, hb1, hw2, hb2)


def kernel(z, pos, batch, edge_index, edge_feats, params):
    p = params
    z = z.astype(jnp.int32)
    ei = edge_index.astype(jnp.int32)
    rowr = ei[0].reshape(NW, ENCH, ECH)
    colr = ei[1].reshape(NW, ENCH, ECH)
    zpad = jnp.pad(z, (0, NPAD - N)).reshape(NW, NNCH, NCH)
    pos16 = jnp.pad(pos.astype(jnp.float32), ((0, 0), (0, DE - 3)))

    h0p, prg, pcg = _sc_prelude(p['emb'], zpad, pos16, rowr, colr)
    h = h0p[:N]
    zeros_acc = jnp.zeros((NACC, D), jnp.float32)
    zb = jnp.zeros((1, D), jnp.float32)

    def edge_layer(l, hA, hB):
        gA, gB = _sc_pair_gather(hA, hB, rowr, colr)
        ew1 = p['g%d_ew1' % l]
        m = _edge_mlp(gA, gB, prg, pcg, edge_feats,
                      ew1[2 * D:2 * D + 1], ew1[2 * D + 1:],
                      p['g%d_eb1' % l].reshape(1, D),
                      p['g%d_ew2' % l], p['g%d_eb2' % l].reshape(1, D),
                      p['g%d_aw' % l].reshape(1, D),
                      p['g%d_ab' % l].reshape(1, 1))
        return _sc_scatter_add(m, rowr, zeros_acc)

    ew1_0, ew1_1 = p['g0_ew1'], p['g1_ew1']
    hA0, hB0 = _mm_multi(h, [ew1_0[:D], ew1_0[D:2 * D]], [zb, zb], 5)
    agg2 = edge_layer(0, hA0, hB0)
    nw1 = p['g0_nw1']
    h, hA1, hB1 = _node_proj(h, agg2[0, :N], agg2[1, :N], nw1[:D], nw1[D:],
                             p['g0_nb1'].reshape(1, D), p['g0_nw2'],
                             p['g0_nb2'].reshape(1, D),
                             [ew1_1[:D], ew1_1[D:2 * D]], [zb, zb])
    agg2 = edge_layer(1, hA1, hB1)
    nw1 = p['g1_nw1']
    h, q, k, v = _node_proj(h, agg2[0, :N], agg2[1, :N], nw1[:D], nw1[D:],
                            p['g1_nb1'].reshape(1, D), p['g1_nw2'],
                            p['g1_nb2'].reshape(1, D),
                            [p['wq'] * 0.25, p['wk'], p['wv']],
                            [p['wq_b'].reshape(1, D) * 0.25,
                             p['wk_b'].reshape(1, D),
                             p['wv_b'].reshape(1, D)])

    padn = ((0, NPAD - N), (0, 0))
    q3 = jnp.pad(q, padn).reshape(NPAD, H, DH).transpose(1, 0, 2)
    k3 = jnp.pad(k, padn).reshape(NPAD, H, DH).transpose(1, 0, 2)
    v3 = jnp.pad(v, padn).reshape(NPAD, H, DH).transpose(1, 0, 2)
    o3 = _attention(q3, k3, v3)
    at = o3.transpose(1, 0, 2).reshape(NPAD, D)[:N]

    return _post_attn_pool(
        h, at, batch.astype(jnp.int32).reshape(5, 1, N // 5),
        p['wo'], p['wo_b'].reshape(1, D),
        p['ln1_g'].reshape(1, D), p['ln1_b'].reshape(1, D),
        p['fw1'], p['fb1'].reshape(1, FFN),
        p['fw2'], p['fb2'].reshape(1, D),
        p['ln2_g'].reshape(1, D), p['ln2_b'].reshape(1, D),
        p['hw1'], p['hb1'].reshape(1, NEURONS),
        p['hw2'], p['hb2'].reshape(1, 1))
